# baseline (device time: 20310 ns/iter reference)
import jax
import jax.numpy as jnp
from jax import lax
from jax.experimental import pallas as pl
from jax.experimental.pallas import tpu as pltpu

N_DEV = 4
B = 2
SQ = 256
D_MODEL = 512
HQ = 4
DH = 64
HD = HQ * DH
BH = B * HQ
SKV_SHARD = 256
BLK = 64
SCALE = 0.125


def kernel(x, Wq, K_ext, V_ext, Wo):
    def body(x_ref, wq_ref, k_ref, v_ref, wo_ref, out_ref,
             ctxbuf, statbuf, csend, crecv, ssend, srecv):
        my = lax.axis_index("i")

        barrier_sem = pltpu.get_barrier_semaphore()
        for j in range(1, N_DEV):
            pl.semaphore_signal(
                barrier_sem, inc=1,
                device_id=((my + j) % N_DEV,),
                device_id_type=pl.DeviceIdType.MESH,
            )
        pl.semaphore_wait(barrier_sem, N_DEV - 1)

        qb = lax.broadcasted_iota(jnp.int32, (SQ, SKV_SHARD), 0) // BLK
        kbg = (my * SKV_SHARD
               + lax.broadcasted_iota(jnp.int32, (SQ, SKV_SHARD), 1)) // BLK
        mask = (qb == kbg) | (kbg == 0) | ((qb + kbg) % 3 == 0)

        wq = wq_ref[...].astype(jnp.bfloat16)

        ctx_rdmas = [[None] * (N_DEV - 1) for _ in range(B)]
        stat_cols = []
        for b in range(B):
            xb = x_ref[b].astype(jnp.bfloat16)
            q = jnp.dot(xb, wq, preferred_element_type=jnp.float32)
            ctx_heads = []
            for h in range(HQ):
                qh = q[:, h * DH:(h + 1) * DH].astype(jnp.bfloat16)
                kT = k_ref[b, h].astype(jnp.bfloat16)
                vT = v_ref[b, h].astype(jnp.bfloat16)
                s = jnp.dot(qh, kT,
                            preferred_element_type=jnp.float32) * SCALE
                s = jnp.where(mask, s, -1e9)
                m = jnp.max(s, axis=1, keepdims=True)
                w = jnp.exp(s - m)
                l = jnp.sum(w, axis=1, keepdims=True)
                ctx_heads.append(
                    lax.dot_general(
                        w.astype(jnp.bfloat16), vT,
                        (((1,), (1,)), ((), ())),
                        preferred_element_type=jnp.float32) / l
                )
                stat_cols.append((m, l))
            ctxbuf[my * B + b] = jnp.concatenate(ctx_heads, axis=1).astype(
                jnp.bfloat16)
            for j in range(N_DEV - 1):
                tgt = (my + 1 + j) % N_DEV
                rc = pltpu.make_async_remote_copy(
                    src_ref=ctxbuf.at[my * B + b],
                    dst_ref=ctxbuf.at[my * B + b],
                    send_sem=csend.at[j * B + b],
                    recv_sem=crecv.at[j * B + b],
                    device_id=(tgt,), device_id_type=pl.DeviceIdType.MESH,
                )
                rc.start()
                ctx_rdmas[b][j] = rc

        m_cols = jnp.concatenate([c[0] for c in stat_cols], axis=1)
        l_cols = jnp.concatenate([c[1] for c in stat_cols], axis=1)
        statbuf[my] = jnp.stack([m_cols.T, l_cols.T])
        stat_rdmas = []
        for j in range(N_DEV - 1):
            tgt = (my + 1 + j) % N_DEV
            rs = pltpu.make_async_remote_copy(
                src_ref=statbuf.at[my], dst_ref=statbuf.at[my],
                send_sem=ssend.at[j], recv_sem=srecv.at[j],
                device_id=(tgt,), device_id_type=pl.DeviceIdType.MESH,
            )
            rs.start()
            stat_rdmas.append(rs)

        for rs in stat_rdmas:
            rs.wait()
        stats = statbuf[...]
        statsT = jnp.transpose(stats, (0, 1, 3, 2))
        m_all = statsT[:, 0]
        l_all = statsT[:, 1]
        M = jnp.max(m_all, axis=0)
        wj = l_all * jnp.exp(m_all - M[None])
        coef = wj / jnp.sum(wj, axis=0)[None]

        S = (lax.broadcasted_iota(jnp.int32, (HQ, HD), 1) // DH
             == lax.broadcasted_iota(jnp.int32, (HQ, HD), 0)
             ).astype(jnp.float32)

        wo = wo_ref[...].astype(jnp.bfloat16)
        for b in range(B):
            for rc in ctx_rdmas[b]:
                rc.wait()
            acc = jnp.zeros((SQ, HD), jnp.float32)
            for slot in range(N_DEV):
                coefw = jnp.dot(coef[slot][:, b * HQ:(b + 1) * HQ], S,
                                preferred_element_type=jnp.float32)
                acc = acc + coefw * ctxbuf[slot * B + b].astype(jnp.float32)
            out_ref[b] = jnp.dot(acc.astype(jnp.bfloat16), wo,
                                 preferred_element_type=jnp.float32
                                 ).astype(jnp.bfloat16)

    kT = jnp.transpose(K_ext, (0, 2, 3, 1))
    vT = jnp.transpose(V_ext, (0, 2, 3, 1))

    return pl.pallas_call(
        body,
        out_shape=jax.ShapeDtypeStruct((B, SQ, D_MODEL), jnp.bfloat16),
        in_specs=[pl.BlockSpec(memory_space=pltpu.VMEM)] * 5,
        out_specs=pl.BlockSpec(memory_space=pltpu.VMEM),
        scratch_shapes=[
            pltpu.VMEM((N_DEV * B, SQ, HD), jnp.bfloat16),
            pltpu.VMEM((N_DEV, 2, BH, SQ), jnp.float32),
            pltpu.SemaphoreType.DMA(((N_DEV - 1) * B,)),
            pltpu.SemaphoreType.DMA(((N_DEV - 1) * B,)),
            pltpu.SemaphoreType.DMA((N_DEV - 1,)),
            pltpu.SemaphoreType.DMA((N_DEV - 1,)),
        ],
        compiler_params=pltpu.CompilerParams(collective_id=0),
    )(x, Wq, kT, vT, Wo)


# device time: 15943 ns/iter; 1.2739x vs baseline; 1.2739x over previous
import jax
import jax.numpy as jnp
from jax import lax
from jax.experimental import pallas as pl
from jax.experimental.pallas import tpu as pltpu

N_DEV = 4
B = 2
SQ = 256
D_MODEL = 512
HQ = 4
DH = 64
HD = HQ * DH
BH = B * HQ
SKV_SHARD = 256
BLK = 64
SCALE = 0.125


def kernel(x, Wq, K_ext, V_ext, Wo):
    def body(x_hbm, wq_hbm, k_hbm, v_hbm, wo_hbm, out_hbm,
             xv, wqv, kv, vv, wov, outv,
             ctxbuf, statbuf, copy_sems, out_sems,
             csend, crecv, ssend, srecv):
        my = lax.axis_index("i")

        cp_x = pltpu.make_async_copy(x_hbm, xv, copy_sems.at[0])
        cp_wq = pltpu.make_async_copy(wq_hbm, wqv, copy_sems.at[1])
        cp_k = pltpu.make_async_copy(k_hbm, kv, copy_sems.at[2])
        cp_v = pltpu.make_async_copy(v_hbm, vv, copy_sems.at[3])
        cp_wo = pltpu.make_async_copy(wo_hbm, wov, copy_sems.at[4])
        for cp in (cp_x, cp_wq, cp_k, cp_v, cp_wo):
            cp.start()

        barrier_sem = pltpu.get_barrier_semaphore()
        for j in range(1, N_DEV):
            pl.semaphore_signal(
                barrier_sem, inc=1,
                device_id=((my + j) % N_DEV,),
                device_id_type=pl.DeviceIdType.MESH,
            )
        pl.semaphore_wait(barrier_sem, N_DEV - 1)

        qb = lax.broadcasted_iota(jnp.int32, (SQ, SKV_SHARD), 0) // BLK
        kbg = (my * SKV_SHARD
               + lax.broadcasted_iota(jnp.int32, (SQ, SKV_SHARD), 1)) // BLK
        mask = (qb == kbg) | (kbg == 0) | ((qb + kbg) % 3 == 0)

        cp_wq.wait()
        cp_x.wait()
        wq = wqv[...].astype(jnp.bfloat16)

        ctx_rdmas = [[None] * (N_DEV - 1) for _ in range(B)]
        stat_cols = []
        for b in range(B):
            xb = xv[b].astype(jnp.bfloat16)
            q = jnp.dot(xb, wq, preferred_element_type=jnp.float32)
            if b == 0:
                cp_k.wait()
                cp_v.wait()
            ctx_heads = []
            for h in range(HQ):
                qh = q[:, h * DH:(h + 1) * DH].astype(jnp.bfloat16)
                kT = kv[b, h].astype(jnp.bfloat16)
                vT = vv[b, h].astype(jnp.bfloat16)
                s = jnp.dot(qh, kT,
                            preferred_element_type=jnp.float32) * SCALE
                s = jnp.where(mask, s, -1e9)
                m = jnp.max(s, axis=1, keepdims=True)
                w = jnp.exp(s - m)
                l = jnp.sum(w, axis=1, keepdims=True)
                ctx_heads.append(
                    lax.dot_general(
                        w.astype(jnp.bfloat16), vT,
                        (((1,), (1,)), ((), ())),
                        preferred_element_type=jnp.float32) / l
                )
                stat_cols.append((m, l))
            ctxbuf[my * B + b] = jnp.concatenate(ctx_heads, axis=1).astype(
                jnp.bfloat16)
            for j in range(N_DEV - 1):
                tgt = (my + 1 + j) % N_DEV
                rc = pltpu.make_async_remote_copy(
                    src_ref=ctxbuf.at[my * B + b],
                    dst_ref=ctxbuf.at[my * B + b],
                    send_sem=csend.at[j * B + b],
                    recv_sem=crecv.at[j * B + b],
                    device_id=(tgt,), device_id_type=pl.DeviceIdType.MESH,
                )
                rc.start()
                ctx_rdmas[b][j] = rc

        m_cols = jnp.concatenate([c[0] for c in stat_cols], axis=1)
        l_cols = jnp.concatenate([c[1] for c in stat_cols], axis=1)
        statbuf[my] = jnp.stack([m_cols.T, l_cols.T])
        stat_rdmas = []
        for j in range(N_DEV - 1):
            tgt = (my + 1 + j) % N_DEV
            rs = pltpu.make_async_remote_copy(
                src_ref=statbuf.at[my], dst_ref=statbuf.at[my],
                send_sem=ssend.at[j], recv_sem=srecv.at[j],
                device_id=(tgt,), device_id_type=pl.DeviceIdType.MESH,
            )
            rs.start()
            stat_rdmas.append(rs)

        for rs in stat_rdmas:
            rs.wait()
        stats = statbuf[...]
        statsT = jnp.transpose(stats, (0, 1, 3, 2))
        m_all = statsT[:, 0]
        l_all = statsT[:, 1]
        M = jnp.max(m_all, axis=0)
        wj = l_all * jnp.exp(m_all - M[None])
        coef = wj / jnp.sum(wj, axis=0)[None]

        S = (lax.broadcasted_iota(jnp.int32, (HQ, HD), 1) // DH
             == lax.broadcasted_iota(jnp.int32, (HQ, HD), 0)
             ).astype(jnp.float32)

        cp_wo.wait()
        wo = wov[...].astype(jnp.bfloat16)
        out_cps = []
        for b in range(B):
            for rc in ctx_rdmas[b]:
                rc.wait()
            acc = jnp.zeros((SQ, HD), jnp.float32)
            for slot in range(N_DEV):
                coefw = jnp.dot(coef[slot][:, b * HQ:(b + 1) * HQ], S,
                                preferred_element_type=jnp.float32)
                acc = acc + coefw * ctxbuf[slot * B + b].astype(jnp.float32)
            outv[b] = jnp.dot(acc.astype(jnp.bfloat16), wo,
                              preferred_element_type=jnp.float32
                              ).astype(jnp.bfloat16)
            cp_o = pltpu.make_async_copy(outv.at[b], out_hbm.at[b],
                                         out_sems.at[b])
            cp_o.start()
            out_cps.append(cp_o)
        for cp_o in out_cps:
            cp_o.wait()

    kT = jnp.transpose(K_ext, (0, 2, 3, 1))
    vT = jnp.transpose(V_ext, (0, 2, 3, 1))

    hbm = pltpu.MemorySpace.HBM
    ins = tuple(pltpu.with_memory_space_constraint(a, hbm)
                for a in (x, Wq, kT, vT, Wo))

    return pl.pallas_call(
        body,
        out_shape=jax.ShapeDtypeStruct((B, SQ, D_MODEL), jnp.bfloat16),
        in_specs=[pl.BlockSpec(memory_space=hbm)] * 5,
        out_specs=pl.BlockSpec(memory_space=hbm),
        scratch_shapes=[
            pltpu.VMEM((B, SQ, D_MODEL), jnp.float32),
            pltpu.VMEM((D_MODEL, HD), jnp.float32),
            pltpu.VMEM((B, HQ, DH, SKV_SHARD), jnp.float32),
            pltpu.VMEM((B, HQ, DH, SKV_SHARD), jnp.float32),
            pltpu.VMEM((HD, D_MODEL), jnp.float32),
            pltpu.VMEM((B, SQ, D_MODEL), jnp.bfloat16),
            pltpu.VMEM((N_DEV * B, SQ, HD), jnp.bfloat16),
            pltpu.VMEM((N_DEV, 2, BH, SQ), jnp.float32),
            pltpu.SemaphoreType.DMA((5,)),
            pltpu.SemaphoreType.DMA((B,)),
            pltpu.SemaphoreType.DMA(((N_DEV - 1) * B,)),
            pltpu.SemaphoreType.DMA(((N_DEV - 1) * B,)),
            pltpu.SemaphoreType.DMA((N_DEV - 1,)),
            pltpu.SemaphoreType.DMA((N_DEV - 1,)),
        ],
        compiler_params=pltpu.CompilerParams(collective_id=0),
    )(*ins)


# device time: 14209 ns/iter; 1.4294x vs baseline; 1.1220x over previous
import jax
import jax.numpy as jnp
from jax import lax
from jax.experimental import pallas as pl
from jax.experimental.pallas import tpu as pltpu

N_DEV = 4
B = 2
SQ = 256
QQ = SQ // N_DEV
D_MODEL = 512
HQ = 4
DH = 64
HD = HQ * DH
SKV_SHARD = 256
BLK = 64
SCALE = 0.125
NP = N_DEV - 1


def kernel(x, Wq, K_ext, V_ext, Wo):
    def body(x_hbm, wq_hbm, k_hbm, v_hbm, wo_hbm, out_hbm,
             xv, wqv, kv, vv, wov,
             myctxq, qctxbuf, mystatq, qstatbuf, outqbuf,
             copy_sems, out_sems,
             qcsend, qcrecv, qssend, qsrecv, oqsend, oqrecv):
        my = lax.axis_index("i")

        cp_x = pltpu.make_async_copy(x_hbm, xv, copy_sems.at[0])
        cp_wq = pltpu.make_async_copy(wq_hbm, wqv, copy_sems.at[1])
        cp_k = pltpu.make_async_copy(k_hbm, kv, copy_sems.at[2])
        cp_v = pltpu.make_async_copy(v_hbm, vv, copy_sems.at[3])
        cp_wo = pltpu.make_async_copy(wo_hbm, wov, copy_sems.at[4])
        for cp in (cp_x, cp_wq, cp_k, cp_v, cp_wo):
            cp.start()

        barrier_sem = pltpu.get_barrier_semaphore()
        for j in range(1, N_DEV):
            pl.semaphore_signal(
                barrier_sem, inc=1,
                device_id=((my + j) % N_DEV,),
                device_id_type=pl.DeviceIdType.MESH,
            )
        pl.semaphore_wait(barrier_sem, N_DEV - 1)

        qb = lax.broadcasted_iota(jnp.int32, (SQ, SKV_SHARD), 0) // BLK
        kbg = (my * SKV_SHARD
               + lax.broadcasted_iota(jnp.int32, (SQ, SKV_SHARD), 1)) // BLK
        mask = (qb == kbg) | (kbg == 0) | ((qb + kbg) % 3 == 0)

        S = (lax.broadcasted_iota(jnp.int32, (HQ, HD), 1) // DH
             == lax.broadcasted_iota(jnp.int32, (HQ, HD), 0)
             ).astype(jnp.float32)

        cp_wq.wait()
        cp_x.wait()
        wq = wqv[...].astype(jnp.bfloat16)

        qc_rdmas = [[None] * NP for _ in range(B)]
        qs_rdmas = [[None] * NP for _ in range(B)]
        for b in range(B):
            xb = xv[b].astype(jnp.bfloat16)
            q = jnp.dot(xb, wq, preferred_element_type=jnp.float32)
            if b == 0:
                cp_k.wait()
                cp_v.wait()
            ctx_heads = []
            stat_cols = []
            for h in range(HQ):
                qh = q[:, h * DH:(h + 1) * DH].astype(jnp.bfloat16)
                kT = kv[b, h].astype(jnp.bfloat16)
                vT = vv[b, h].astype(jnp.bfloat16)
                s = jnp.dot(qh, kT,
                            preferred_element_type=jnp.float32) * SCALE
                s = jnp.where(mask, s, -1e9)
                m = jnp.max(s, axis=1, keepdims=True)
                w = jnp.exp(s - m)
                l = jnp.sum(w, axis=1, keepdims=True)
                ctx_heads.append(
                    lax.dot_general(
                        w.astype(jnp.bfloat16), vT,
                        (((1,), (1,)), ((), ())),
                        preferred_element_type=jnp.float32) / l
                )
                stat_cols.append((m, l))
            ctx_b = jnp.concatenate(ctx_heads, axis=1).astype(jnp.bfloat16)
            m_b = jnp.concatenate([c[0] for c in stat_cols], axis=1).T
            l_b = jnp.concatenate([c[1] for c in stat_cols], axis=1).T
            st_b = jnp.stack([m_b, l_b])
            for qr in range(N_DEV):
                myctxq[qr * B + b] = ctx_b[qr * QQ:(qr + 1) * QQ, :]
                mystatq[qr * B + b] = st_b[:, :, qr * QQ:(qr + 1) * QQ]
            qctxbuf[my * B + b] = myctxq[my * B + b]
            qstatbuf[my * B + b] = mystatq[my * B + b]
            for j in range(NP):
                tgt = (my + 1 + j) % N_DEV
                rc = pltpu.make_async_remote_copy(
                    src_ref=myctxq.at[tgt * B + b],
                    dst_ref=qctxbuf.at[my * B + b],
                    send_sem=qcsend.at[j * B + b],
                    recv_sem=qcrecv.at[j * B + b],
                    device_id=(tgt,), device_id_type=pl.DeviceIdType.MESH,
                )
                rs = pltpu.make_async_remote_copy(
                    src_ref=mystatq.at[tgt * B + b],
                    dst_ref=qstatbuf.at[my * B + b],
                    send_sem=qssend.at[j * B + b],
                    recv_sem=qsrecv.at[j * B + b],
                    device_id=(tgt,), device_id_type=pl.DeviceIdType.MESH,
                )
                rc.start()
                rs.start()
                qc_rdmas[b][j] = rc
                qs_rdmas[b][j] = rs

        cp_wo.wait()
        wo = wov[...].astype(jnp.bfloat16)
        oq_rdmas = [[None] * NP for _ in range(B)]
        own_cps = []
        for b in range(B):
            for j in range(NP):
                qc_rdmas[b][j].wait()
                qs_rdmas[b][j].wait()
            ms = [qstatbuf[o * B + b, 0] for o in range(N_DEV)]
            ls = [qstatbuf[o * B + b, 1] for o in range(N_DEV)]
            M = jnp.maximum(jnp.maximum(ms[0], ms[1]),
                            jnp.maximum(ms[2], ms[3]))
            wjs = [ls[o] * jnp.exp(ms[o] - M) for o in range(N_DEV)]
            den = (wjs[0] + wjs[1]) + (wjs[2] + wjs[3])
            acc = jnp.zeros((QQ, HD), jnp.float32)
            for o in range(N_DEV):
                coefw = lax.dot_general(
                    wjs[o] / den, S, (((0,), (0,)), ((), ())),
                    preferred_element_type=jnp.float32)
                acc = acc + coefw * qctxbuf[o * B + b].astype(jnp.float32)
            outqbuf[my * B + b] = jnp.dot(
                acc.astype(jnp.bfloat16), wo,
                preferred_element_type=jnp.float32).astype(jnp.bfloat16)
            cp = pltpu.make_async_copy(
                outqbuf.at[my * B + b],
                out_hbm.at[b, pl.ds(my * QQ, QQ)],
                out_sems.at[NP * B + b])
            cp.start()
            own_cps.append(cp)
            for j in range(NP):
                tgt = (my + 1 + j) % N_DEV
                ro = pltpu.make_async_remote_copy(
                    src_ref=outqbuf.at[my * B + b],
                    dst_ref=outqbuf.at[my * B + b],
                    send_sem=oqsend.at[j * B + b],
                    recv_sem=oqrecv.at[j * B + b],
                    device_id=(tgt,), device_id_type=pl.DeviceIdType.MESH,
                )
                ro.start()
                oq_rdmas[b][j] = ro

        out_cps = list(own_cps)
        for b in range(B):
            for j in range(NP):
                oq_rdmas[b][j].wait()
                src = (my - 1 - j) % N_DEV
                cp = pltpu.make_async_copy(
                    outqbuf.at[src * B + b],
                    out_hbm.at[b, pl.ds(src * QQ, QQ)],
                    out_sems.at[j * B + b])
                cp.start()
                out_cps.append(cp)
        for cp in out_cps:
            cp.wait()

    kT = jnp.transpose(K_ext, (0, 2, 3, 1))
    vT = jnp.transpose(V_ext, (0, 2, 3, 1))

    hbm = pltpu.MemorySpace.HBM
    ins = tuple(pltpu.with_memory_space_constraint(a, hbm)
                for a in (x, Wq, kT, vT, Wo))

    return pl.pallas_call(
        body,
        out_shape=jax.ShapeDtypeStruct((B, SQ, D_MODEL), jnp.bfloat16),
        in_specs=[pl.BlockSpec(memory_space=hbm)] * 5,
        out_specs=pl.BlockSpec(memory_space=hbm),
        scratch_shapes=[
            pltpu.VMEM((B, SQ, D_MODEL), jnp.float32),
            pltpu.VMEM((D_MODEL, HD), jnp.float32),
            pltpu.VMEM((B, HQ, DH, SKV_SHARD), jnp.float32),
            pltpu.VMEM((B, HQ, DH, SKV_SHARD), jnp.float32),
            pltpu.VMEM((HD, D_MODEL), jnp.float32),
            pltpu.VMEM((N_DEV * B, QQ, HD), jnp.bfloat16),
            pltpu.VMEM((N_DEV * B, QQ, HD), jnp.bfloat16),
            pltpu.VMEM((N_DEV * B, 2, HQ, QQ), jnp.float32),
            pltpu.VMEM((N_DEV * B, 2, HQ, QQ), jnp.float32),
            pltpu.VMEM((N_DEV * B, QQ, D_MODEL), jnp.bfloat16),
            pltpu.SemaphoreType.DMA((5,)),
            pltpu.SemaphoreType.DMA((N_DEV * B,)),
            pltpu.SemaphoreType.DMA((NP * B,)),
            pltpu.SemaphoreType.DMA((NP * B,)),
            pltpu.SemaphoreType.DMA((NP * B,)),
            pltpu.SemaphoreType.DMA((NP * B,)),
            pltpu.SemaphoreType.DMA((NP * B,)),
            pltpu.SemaphoreType.DMA((NP * B,)),
        ],
        compiler_params=pltpu.CompilerParams(collective_id=0),
    )(*ins)


# device time: 13294 ns/iter; 1.5278x vs baseline; 1.0688x over previous
import jax
import jax.numpy as jnp
from jax import lax
from jax.experimental import pallas as pl
from jax.experimental.pallas import tpu as pltpu

N_DEV = 4
B = 2
SQ = 256
QQ = SQ // N_DEV
D_MODEL = 512
HQ = 4
DH = 64
HD = HQ * DH
SKV_SHARD = 256
BLK = 64
SCALE = 0.125
NP = N_DEV - 1


def kernel(x, Wq, K_ext, V_ext, Wo):
    def body(x_hbm, wq_hbm, k_hbm, v_hbm, wo_hbm, out_hbm,
             xv, wqv, kv, vv, wov, outv,
             myctxq, qctxbuf, mystatq, qstatbuf, gathbuf,
             copy_sems, out_sems,
             qcsend, qcrecv, qssend, qsrecv, oqsend, oqrecv):
        my = lax.axis_index("i")

        cp_x = pltpu.make_async_copy(x_hbm, xv, copy_sems.at[0])
        cp_wq = pltpu.make_async_copy(wq_hbm, wqv, copy_sems.at[1])
        cp_k = pltpu.make_async_copy(k_hbm, kv, copy_sems.at[2])
        cp_v = pltpu.make_async_copy(v_hbm, vv, copy_sems.at[3])
        cp_wo = pltpu.make_async_copy(wo_hbm, wov, copy_sems.at[4])
        for cp in (cp_x, cp_wq, cp_k, cp_v, cp_wo):
            cp.start()

        barrier_sem = pltpu.get_barrier_semaphore()
        for j in range(1, N_DEV):
            pl.semaphore_signal(
                barrier_sem, inc=1,
                device_id=((my + j) % N_DEV,),
                device_id_type=pl.DeviceIdType.MESH,
            )
        pl.semaphore_wait(barrier_sem, N_DEV - 1)

        qb = lax.broadcasted_iota(jnp.int32, (SQ, SKV_SHARD), 0) // BLK
        kbg = (my * SKV_SHARD
               + lax.broadcasted_iota(jnp.int32, (SQ, SKV_SHARD), 1)) // BLK
        mask = (qb == kbg) | (kbg == 0) | ((qb + kbg) % 3 == 0)

        S = (lax.broadcasted_iota(jnp.int32, (HQ, HD), 1) // DH
             == lax.broadcasted_iota(jnp.int32, (HQ, HD), 0)
             ).astype(jnp.float32)

        cp_wq.wait()
        cp_x.wait()
        wq = wqv[...].astype(jnp.bfloat16)

        qc_rdmas = [[None] * NP for _ in range(B)]
        qs_rdmas = [[None] * NP for _ in range(B)]
        for b in range(B):
            xb = xv[b].astype(jnp.bfloat16)
            q = jnp.dot(xb, wq, preferred_element_type=jnp.float32)
            if b == 0:
                cp_k.wait()
                cp_v.wait()
            ctx_heads = []
            stat_cols = []
            for h in range(HQ):
                qh = q[:, h * DH:(h + 1) * DH].astype(jnp.bfloat16)
                kT = kv[b, h].astype(jnp.bfloat16)
                vT = vv[b, h].astype(jnp.bfloat16)
                s = jnp.dot(qh, kT,
                            preferred_element_type=jnp.float32) * SCALE
                s = jnp.where(mask, s, -1e9)
                m = jnp.max(s, axis=1, keepdims=True)
                w = jnp.exp(s - m)
                l = jnp.sum(w, axis=1, keepdims=True)
                ctx_heads.append(
                    lax.dot_general(
                        w.astype(jnp.bfloat16), vT,
                        (((1,), (1,)), ((), ())),
                        preferred_element_type=jnp.float32) / l
                )
                stat_cols.append((m, l))
            ctx_b = jnp.concatenate(ctx_heads, axis=1).astype(jnp.bfloat16)
            m_b = jnp.concatenate([c[0] for c in stat_cols], axis=1).T
            l_b = jnp.concatenate([c[1] for c in stat_cols], axis=1).T
            st_b = jnp.stack([m_b, l_b])
            for qr in range(N_DEV):
                myctxq[qr * B + b] = ctx_b[qr * QQ:(qr + 1) * QQ, :]
                mystatq[qr * B + b] = st_b[:, :, qr * QQ:(qr + 1) * QQ]
            qctxbuf[my * B + b] = myctxq[my * B + b]
            qstatbuf[my * B + b] = mystatq[my * B + b]
            for j in range(NP):
                tgt = (my + 1 + j) % N_DEV
                rc = pltpu.make_async_remote_copy(
                    src_ref=myctxq.at[tgt * B + b],
                    dst_ref=qctxbuf.at[my * B + b],
                    send_sem=qcsend.at[j * B + b],
                    recv_sem=qcrecv.at[j * B + b],
                    device_id=(tgt,), device_id_type=pl.DeviceIdType.MESH,
                )
                rs = pltpu.make_async_remote_copy(
                    src_ref=mystatq.at[tgt * B + b],
                    dst_ref=qstatbuf.at[my * B + b],
                    send_sem=qssend.at[j * B + b],
                    recv_sem=qsrecv.at[j * B + b],
                    device_id=(tgt,), device_id_type=pl.DeviceIdType.MESH,
                )
                rc.start()
                rs.start()
                qc_rdmas[b][j] = rc
                qs_rdmas[b][j] = rs

        oq_rdmas = [[None] * NP for _ in range(B)]
        for b in range(B):
            for j in range(NP):
                qc_rdmas[b][j].wait()
                qs_rdmas[b][j].wait()
            ms = [qstatbuf[o * B + b, 0] for o in range(N_DEV)]
            ls = [qstatbuf[o * B + b, 1] for o in range(N_DEV)]
            M = jnp.maximum(jnp.maximum(ms[0], ms[1]),
                            jnp.maximum(ms[2], ms[3]))
            wjs = [ls[o] * jnp.exp(ms[o] - M) for o in range(N_DEV)]
            den = (wjs[0] + wjs[1]) + (wjs[2] + wjs[3])
            acc = jnp.zeros((QQ, HD), jnp.float32)
            for o in range(N_DEV):
                coefw = lax.dot_general(
                    wjs[o] / den, S, (((0,), (0,)), ((), ())),
                    preferred_element_type=jnp.float32)
                acc = acc + coefw * qctxbuf[o * B + b].astype(jnp.float32)
            gathbuf[my * B + b] = acc.astype(jnp.bfloat16)
            for j in range(NP):
                tgt = (my + 1 + j) % N_DEV
                ro = pltpu.make_async_remote_copy(
                    src_ref=gathbuf.at[my * B + b],
                    dst_ref=gathbuf.at[my * B + b],
                    send_sem=oqsend.at[j * B + b],
                    recv_sem=oqrecv.at[j * B + b],
                    device_id=(tgt,), device_id_type=pl.DeviceIdType.MESH,
                )
                ro.start()
                oq_rdmas[b][j] = ro

        cp_wo.wait()
        wo = wov[...].astype(jnp.bfloat16)
        out_cps = []
        for b in range(B):
            for j in range(NP):
                oq_rdmas[b][j].wait()
            ctx_full = jnp.concatenate(
                [gathbuf[o * B + b] for o in range(N_DEV)], axis=0)
            outv[b] = jnp.dot(ctx_full, wo,
                              preferred_element_type=jnp.float32
                              ).astype(jnp.bfloat16)
            cp = pltpu.make_async_copy(outv.at[b], out_hbm.at[b],
                                       out_sems.at[b])
            cp.start()
            out_cps.append(cp)
        for cp in out_cps:
            cp.wait()

    kT = jnp.transpose(K_ext, (0, 2, 3, 1))
    vT = jnp.transpose(V_ext, (0, 2, 3, 1))

    hbm = pltpu.MemorySpace.HBM
    ins = tuple(pltpu.with_memory_space_constraint(a, hbm)
                for a in (x, Wq, kT, vT, Wo))

    return pl.pallas_call(
        body,
        out_shape=jax.ShapeDtypeStruct((B, SQ, D_MODEL), jnp.bfloat16),
        in_specs=[pl.BlockSpec(memory_space=hbm)] * 5,
        out_specs=pl.BlockSpec(memory_space=hbm),
        scratch_shapes=[
            pltpu.VMEM((B, SQ, D_MODEL), jnp.float32),
            pltpu.VMEM((D_MODEL, HD), jnp.float32),
            pltpu.VMEM((B, HQ, DH, SKV_SHARD), jnp.float32),
            pltpu.VMEM((B, HQ, DH, SKV_SHARD), jnp.float32),
            pltpu.VMEM((HD, D_MODEL), jnp.float32),
            pltpu.VMEM((B, SQ, D_MODEL), jnp.bfloat16),
            pltpu.VMEM((N_DEV * B, QQ, HD), jnp.bfloat16),
            pltpu.VMEM((N_DEV * B, QQ, HD), jnp.bfloat16),
            pltpu.VMEM((N_DEV * B, 2, HQ, QQ), jnp.float32),
            pltpu.VMEM((N_DEV * B, 2, HQ, QQ), jnp.float32),
            pltpu.VMEM((N_DEV * B, QQ, HD), jnp.bfloat16),
            pltpu.SemaphoreType.DMA((5,)),
            pltpu.SemaphoreType.DMA((B,)),
            pltpu.SemaphoreType.DMA((NP * B,)),
            pltpu.SemaphoreType.DMA((NP * B,)),
            pltpu.SemaphoreType.DMA((NP * B,)),
            pltpu.SemaphoreType.DMA((NP * B,)),
            pltpu.SemaphoreType.DMA((NP * B,)),
            pltpu.SemaphoreType.DMA((NP * B,)),
        ],
        compiler_params=pltpu.CompilerParams(collective_id=0),
    )(*ins)
